# baseline (speedup 1.0000x reference)
import functools

import jax
import jax.numpy as jnp
from jax import lax
from jax.experimental import pallas as pl
from jax.experimental.pallas import tpu as pltpu
from jax.experimental.pallas import tpu_sc as plsc

NUM_EMBEDDINGS = 100000
EMBED_DIM = 64
PADDING_IDX = NUM_EMBEDDINGS * 4

_LANES = 16
_NW = 32
_CHUNK = 800


def _make_lookup(n):
    assert n % (_NW * _CHUNK * 4) == 0
    per_w = n // _NW
    steps = per_w // _CHUNK
    pairs = steps // 2
    mesh = plsc.VectorSubcoreMesh(core_axis_name="c", subcore_axis_name="s")

    @functools.partial(
        pl.kernel,
        mesh=mesh,
        out_type=jax.ShapeDtypeStruct((n, EMBED_DIM), jnp.float32),
        scratch_types=[
            pltpu.VMEM((2, 3, 2 * _CHUNK), jnp.int32),
            pltpu.VMEM((2, _CHUNK), jnp.int32),
            pltpu.VMEM((2, _CHUNK), jnp.float32),
            pltpu.VMEM((2, _CHUNK, EMBED_DIM), jnp.float32),
            pltpu.SemaphoreType.DMA,
            pltpu.SemaphoreType.DMA,
            pltpu.SemaphoreType.DMA,
            pltpu.SemaphoreType.DMA,
            pltpu.SemaphoreType.DMA,
            pltpu.SemaphoreType.DMA,
        ],
        compiler_params=pltpu.CompilerParams(use_tc_tiling_on_sc=False),
    )
    def lookup(pk_h, tab_h, out_h,
               st_v, idx_v, mkf_v, rows_v,
               in_s0, in_s1, g_s0, g_s1, w_s0, w_s1):
        in_s = [in_s0, in_s1]
        g_s = [g_s0, g_s1]
        w_s = [w_s0, w_s1]
        wid = lax.axis_index("s") * 2 + lax.axis_index("c")
        wbase = wid * per_w

        def fire_pair_inputs(t, p):
            base = wbase + t * (2 * _CHUNK)
            pltpu.async_copy(
                pk_h.at[:, pl.ds(base, 2 * _CHUNK)], st_v.at[p], in_s[p])

        def drain_pair_inputs(p):
            pltpu.make_async_copy(
                pk_h.at[:, pl.ds(wbase, 2 * _CHUNK)], st_v.at[p], in_s[p]).wait()

        def compute_idx(p, b):
            off = b * _CHUNK

            @plsc.parallel_loop(0, _CHUNK // _LANES)
            def cbody(i):
                src = pl.ds(off + i * _LANES, _LANES)
                dst = pl.ds(i * _LANES, _LANES)
                idx_v[b, dst] = st_v[p, 0, src] * NUM_EMBEDDINGS + st_v[p, 1, src]
                mkf_v[b, dst] = st_v[p, 2, src].astype(jnp.float32)

        def zero_masked(b):
            @plsc.parallel_loop(0, _CHUNK // _LANES)
            def zbody(k):
                m16 = mkf_v[b, pl.ds(k * _LANES, _LANES)]
                for j in range(_LANES):
                    r = k * _LANES + j
                    m = m16.at[jnp.full((_LANES,), j, jnp.int32)].get(
                        mode="promise_in_bounds")
                    for q in range(EMBED_DIM // _LANES):
                        sl = pl.ds(q * _LANES, _LANES)
                        rows_v[b, r, sl] = rows_v[b, r, sl] * m

        def fire_gather(b):
            pltpu.async_copy(tab_h.at[idx_v.at[b]], rows_v.at[b], g_s[b])

        def drain_gather(b):
            pltpu.make_async_copy(
                tab_h.at[idx_v.at[b]], rows_v.at[b], g_s[b]).wait()

        def fire_wb(g, b):
            base = wbase + g * _CHUNK
            pltpu.async_copy(rows_v.at[b], out_h.at[pl.ds(base, _CHUNK)], w_s[b])

        def drain_wb(b):
            pltpu.make_async_copy(
                rows_v.at[b], out_h.at[pl.ds(wbase, _CHUNK)], w_s[b]).wait()

        def half_step(g, p, b):
            b1 = 1 - b
            compute_idx(p, b)
            drain_wb(b)
            fire_gather(b)
            drain_gather(b1)
            zero_masked(b1)
            fire_wb(g - 1, b1)

        def pair(t, p):
            drain_pair_inputs(p)
            tnext = jnp.where(t + 1 < pairs, t + 1, 0)
            fire_pair_inputs(tnext, 1 - p)
            half_step(2 * t, p, 0)
            half_step(2 * t + 1, p, 1)

        fire_pair_inputs(0, 0)
        drain_pair_inputs(0)
        fire_pair_inputs(1, 1)
        compute_idx(0, 0)
        fire_gather(0)
        compute_idx(0, 1)
        fire_gather(1)
        drain_gather(0)
        zero_masked(0)
        fire_wb(0, 0)
        pair(1, 1)

        def body(u, carry):
            pair(2 * u, 0)
            pair(2 * u + 1, 1)
            return carry

        lax.fori_loop(1, pairs // 2, body, 0)

        drain_gather(1)
        zero_masked(1)
        fire_wb(steps - 1, 1)
        drain_wb(0)
        drain_wb(1)
        drain_pair_inputs(0)

    return lookup


def kernel(token_type_ids, sem_ids, seq_mask, emb_weight):
    b, l = token_type_ids.shape
    n = b * l
    packed = jnp.stack([
        token_type_ids.reshape(n).astype(jnp.int32),
        sem_ids.reshape(n).astype(jnp.int32),
        seq_mask.reshape(n).astype(jnp.int32),
    ])
    out = _make_lookup(n)(packed, emb_weight)
    return out.reshape(b, l, EMBED_DIM)
